# Initial kernel scaffold; baseline (speedup 1.0000x reference)
import functools

import jax
import jax.numpy as jnp
from jax import lax
from jax.experimental import pallas as pl
from jax.experimental.pallas import tpu as pltpu
from jax.experimental.pallas import tpu_sc as plsc

N = 10000
E = 320000
NFEAT = 128
NHID = 32
NCLASS = 16
BATCH = 1024

NPAD = 10240
DUMMY = N
NC = 2
NS = 16
NW = NC * NS
CHUNK = 128
CPW = 79
EPW = CPW * CHUNK
EPAD = EPW * NW
RPT = NPAD // NS
KB = 2000


def _mesh():
    return plsc.VectorSubcoreMesh(
        core_axis_name="c", subcore_axis_name="s", num_cores=NC, num_subcores=NS
    )


def _zero_rows(ref, nrows, width):
    z = jnp.zeros((16,), jnp.float32)

    def body(i, _):
        for h in range(width // 16):
            ref[i, pl.ds(16 * h, 16)] = z
        return 0

    lax.fori_loop(0, nrows, body, 0)


def _zero_flat(ref, nwords):
    z = jnp.zeros((16,), jnp.float32)

    def body(i, _):
        ref[pl.ds(16 * i, 16)] = z
        return 0

    lax.fori_loop(0, nwords // 16, body, 0)


def _deg_body(idx_hbm, out_hbm, d0, d1, d2, d3, idx_v, ones_v, zero_v):
    c = lax.axis_index("c")
    s = lax.axis_index("s")
    wid = c * NS + s
    degs = [d0, d1, d2, d3]

    for h in range(CHUNK // 16):
        ones_v[pl.ds(16 * h, 16)] = jnp.ones((16,), jnp.float32)
    _zero_flat(zero_v, RPT)
    for k in range(4):
        pltpu.sync_copy(zero_v, degs[k].at[pl.ds(s * RPT, RPT)])
    plsc.subcore_barrier()

    for k in range(4):
        pltpu.sync_copy(idx_hbm.at[k, wid], idx_v)

        def chunk(j, _, k=k):
            pltpu.sync_copy(ones_v, degs[k].at[idx_v.at[j]], add=True)
            return 0

        lax.fori_loop(0, CPW, chunk, 0)
    plsc.subcore_barrier()

    for k in range(4):
        pltpu.sync_copy(
            degs[k].at[pl.ds(s * RPT, RPT)], out_hbm.at[c, k, pl.ds(s * RPT, RPT)]
        )


def _deg_call(idx_all):
    f = pl.kernel(
        _deg_body,
        out_type=jax.ShapeDtypeStruct((NC, 4, NPAD), jnp.float32),
        mesh=_mesh(),
        scratch_types=[
            pltpu.VMEM_SHARED((NPAD,), jnp.float32),
            pltpu.VMEM_SHARED((NPAD,), jnp.float32),
            pltpu.VMEM_SHARED((NPAD,), jnp.float32),
            pltpu.VMEM_SHARED((NPAD,), jnp.float32),
            pltpu.VMEM((CPW, CHUNK), jnp.int32),
            pltpu.VMEM((CHUNK,), jnp.float32),
            pltpu.VMEM((RPT,), jnp.float32),
        ],
        name="gcn_degrees_sc",
    )
    return f(idx_all)


def _msg_body(D, idx_hbm, table_hbm, out_hbm, a0, a1, src_v, dst_v, rows_v,
              zero_v, sem):
    c = lax.axis_index("c")
    s = lax.axis_index("s")
    wid = c * NS + s
    aggs = [a0, a1]

    _zero_rows(zero_v, RPT, D)
    for g in range(2):
        pltpu.sync_copy(zero_v, aggs[g].at[pl.ds(s * RPT, RPT)])
    plsc.subcore_barrier()

    for g in range(2):
        pltpu.sync_copy(idx_hbm.at[2 * g, wid], src_v)
        pltpu.sync_copy(idx_hbm.at[2 * g + 1, wid], dst_v)

        def chunk(j, _, g=g):
            pltpu.async_copy(table_hbm.at[g].at[src_v.at[j]], rows_v, sem).wait()
            pltpu.sync_copy(rows_v, aggs[g].at[dst_v.at[j]], add=True)
            return 0

        lax.fori_loop(0, CPW, chunk, 0)
    plsc.subcore_barrier()

    for g in range(2):
        pltpu.sync_copy(
            aggs[g].at[pl.ds(s * RPT, RPT)], out_hbm.at[c, g, pl.ds(s * RPT, RPT)]
        )


def _msg_call(idx_all, table, D, tag):
    f = pl.kernel(
        functools.partial(_msg_body, D),
        out_type=jax.ShapeDtypeStruct((NC, 2, NPAD, D), jnp.float32),
        mesh=_mesh(),
        scratch_types=[
            pltpu.VMEM_SHARED((NPAD, D), jnp.float32),
            pltpu.VMEM_SHARED((NPAD, D), jnp.float32),
            pltpu.VMEM((CPW, CHUNK), jnp.int32),
            pltpu.VMEM((CPW, CHUNK), jnp.int32),
            pltpu.VMEM((CHUNK, D), jnp.float32),
            pltpu.VMEM((RPT, D), jnp.float32),
            pltpu.SemaphoreType.DMA,
        ],
        name=f"gcn_msgpass_{tag}_sc",
    )
    return f(idx_all, table)


def _projA_body(sux, sox, w1, degp, table_out, norms_out):
    deg = degp[0] + degp[1]
    norms = jnp.where(deg > 0.0, lax.rsqrt(jnp.maximum(deg, 1e-30)), 0.0)
    norms_out[...] = norms
    h_su = jnp.dot(sux[...], w1[...], preferred_element_type=jnp.float32)
    h_so = jnp.dot(sox[...], w1[...], preferred_element_type=jnp.float32)
    table_out[0] = h_su * norms[0][:, None]
    table_out[1] = h_so * norms[2][:, None]


def _projA_call(sux_pad, sox_pad, w1, deg_partials):
    return pl.pallas_call(
        _projA_body,
        out_shape=(
            jax.ShapeDtypeStruct((2, NPAD, NHID), jnp.float32),
            jax.ShapeDtypeStruct((4, NPAD), jnp.float32),
        ),
        name="gcn_proj1_tc",
    )(sux_pad, sox_pad, w1, deg_partials)


def _projB_body(agg1, norms, b1, w2, f1_out, table2_out):
    for g in range(2):
        agg = agg1[0, g] + agg1[1, g]
        f1 = jnp.maximum(agg * norms[2 * g + 1][:, None] + b1[...][None, :], 0.0)
        f1_out[g] = f1
        h2 = jnp.dot(f1, w2[...], preferred_element_type=jnp.float32)
        table2_out[g] = h2 * norms[2 * g][:, None]


def _projB_call(agg1, norms, b1, w2):
    return pl.pallas_call(
        _projB_body,
        out_shape=(
            jax.ShapeDtypeStruct((2, NPAD, NHID), jnp.float32),
            jax.ShapeDtypeStruct((2, NPAD, NCLASS), jnp.float32),
        ),
        name="gcn_proj2_tc",
    )(agg1, norms, b1, w2)


def _pool_body(l_su, l_so, f1, agg2, norms, b2, fc1w, fc1b, fc2w, fc2b, fc3w,
               fc3b, out, acc, rs):
    k = pl.program_id(0)
    nk = pl.num_programs(0)

    @pl.when(k == 0)
    def _():
        acc[...] = jnp.zeros_like(acc)
        rs[...] = jnp.zeros_like(rs)

    nrm = norms[...]
    f_su = (agg2[0, 0] + agg2[1, 0]) * nrm[1][:, None]
    f_so = (agg2[0, 1] + agg2[1, 1]) * nrm[3][:, None]
    cat_su = jnp.concatenate([f1[0], f_su], axis=1)
    cat_so = jnp.concatenate([f1[1], f_so], axis=1)
    lsu = l_su[...]
    lso = l_so[...]
    acc[:, 0:48] += jnp.dot(lsu, cat_su, preferred_element_type=jnp.float32)
    acc[:, 48:96] += jnp.dot(lso, cat_so, preferred_element_type=jnp.float32)
    rs[:, 0] += jnp.sum(lsu, axis=1)
    rs[:, 1] += jnp.sum(lso, axis=1)

    @pl.when(k == nk - 1)
    def _():
        data = acc[...]
        b2v = b2[...]
        corr = jnp.concatenate(
            [
                jnp.zeros((BATCH, NHID), jnp.float32),
                rs[:, 0][:, None] * b2v[None, :],
                jnp.zeros((BATCH, NHID), jnp.float32),
                rs[:, 1][:, None] * b2v[None, :],
            ],
            axis=1,
        )
        data = data + corr
        d1 = jnp.maximum(jnp.dot(data, fc1w[...], preferred_element_type=jnp.float32)
                         + fc1b[...][None, :], 0.0)
        d2 = jnp.maximum(jnp.dot(d1, fc2w[...], preferred_element_type=jnp.float32)
                         + fc2b[...][None, :], 0.0)
        out[...] = jnp.dot(d2, fc3w[...], preferred_element_type=jnp.float32) \
            + fc3b[...][None, :]


def _pool_call(l_su, l_so, f1, agg2, norms, b2, fcs):
    nblk = N // KB
    in_specs = [
        pl.BlockSpec((BATCH, KB), lambda k: (0, k)),
        pl.BlockSpec((BATCH, KB), lambda k: (0, k)),
        pl.BlockSpec((2, KB, NHID), lambda k: (0, k, 0)),
        pl.BlockSpec((NC, 2, KB, NCLASS), lambda k: (0, 0, k, 0)),
        pl.BlockSpec((4, KB), lambda k: (0, k)),
        pl.BlockSpec((NCLASS,), lambda k: (0,)),
        pl.BlockSpec((96, 64), lambda k: (0, 0)),
        pl.BlockSpec((64,), lambda k: (0,)),
        pl.BlockSpec((64, 16), lambda k: (0, 0)),
        pl.BlockSpec((16,), lambda k: (0,)),
        pl.BlockSpec((16, 1), lambda k: (0, 0)),
        pl.BlockSpec((1,), lambda k: (0,)),
    ]
    return pl.pallas_call(
        _pool_body,
        grid=(nblk,),
        in_specs=in_specs,
        out_specs=pl.BlockSpec((BATCH, 1), lambda k: (0, 0)),
        out_shape=jax.ShapeDtypeStruct((BATCH, 1), jnp.float32),
        scratch_shapes=[
            pltpu.VMEM((BATCH, 96), jnp.float32),
            pltpu.VMEM((BATCH, 2), jnp.float32),
        ],
        name="gcn_pool_mlp_tc",
    )(l_su, l_so, f1, agg2, norms, b2, *fcs)


def _prep_idx(edge_index):
    pad = jnp.full((2, EPAD - E), DUMMY, jnp.int32)
    idx = jnp.concatenate([edge_index.astype(jnp.int32), pad], axis=1)
    return idx.reshape(2, NW, CPW, CHUNK)


def kernel(solute_x, solute_edge_index, solvent_x, solvent_edge_index,
           solute_len_matrix, solvent_len_matrix, W1, b1, W2, b2,
           fc1_W, fc1_b, fc2_W, fc2_b, fc3_W, fc3_b):
    idx_all = jnp.concatenate(
        [_prep_idx(solute_edge_index), _prep_idx(solvent_edge_index)], axis=0
    )
    sux = jnp.pad(solute_x, ((0, NPAD - N), (0, 0)))
    sox = jnp.pad(solvent_x, ((0, NPAD - N), (0, 0)))

    deg_partials = _deg_call(idx_all)
    table1, norms = _projA_call(sux, sox, W1, deg_partials)
    agg1 = _msg_call(idx_all, table1, NHID, "l1")
    f1, table2 = _projB_call(agg1, norms, b1, W2)
    agg2 = _msg_call(idx_all, table2, NCLASS, "l2")

    f1_t = f1[:, :N, :]
    agg2_t = agg2[:, :, :N, :]
    norms_t = norms[:, :N]
    fcs = (fc1_W, fc1_b, fc2_W, fc2_b, fc3_W, fc3_b)
    return _pool_call(solute_len_matrix, solvent_len_matrix, f1_t, agg2_t,
                      norms_t, b2, fcs)


# speedup vs baseline: 12.9908x; 12.9908x over previous
import functools

import jax
import jax.numpy as jnp
from jax import lax
from jax.experimental import pallas as pl
from jax.experimental.pallas import tpu as pltpu
from jax.experimental.pallas import tpu_sc as plsc

N = 10000
E = 320000
NFEAT = 128
NHID = 32
NCLASS = 16
BATCH = 1024

NPAD = 10240
DUMMY = N
NC = 2
NS = 16
NW = NC * NS
CHUNK = 128
CPW = 79
EPW = CPW * CHUNK
EPAD = EPW * NW
RPT = NPAD // NS
KB = 2000


def _mesh():
    return plsc.VectorSubcoreMesh(
        core_axis_name="c", subcore_axis_name="s", num_cores=NC, num_subcores=NS
    )


def _zero_rows(ref, nrows, width):
    z = jnp.zeros((16,), jnp.float32)

    def body(i, _):
        for h in range(width // 16):
            ref[i, pl.ds(16 * h, 16)] = z
        return 0

    lax.fori_loop(0, nrows, body, 0)


def _zero_flat(ref, nwords):
    z = jnp.zeros((16,), jnp.float32)

    def body(i, _):
        ref[pl.ds(16 * i, 16)] = z
        return 0

    lax.fori_loop(0, nwords // 16, body, 0)


def _deg_body(idx_hbm, out_hbm, d0, d1, d2, d3, idx_v, ones_v, zero_v):
    c = lax.axis_index("c")
    s = lax.axis_index("s")
    wid = c * NS + s
    degs = [d0, d1, d2, d3]

    for h in range(CHUNK // 16):
        ones_v[pl.ds(16 * h, 16)] = jnp.ones((16,), jnp.float32)
    _zero_flat(zero_v, RPT)
    for k in range(4):
        pltpu.sync_copy(zero_v, degs[k].at[pl.ds(s * RPT, RPT)])
    plsc.subcore_barrier()

    for k in range(4):
        pltpu.sync_copy(idx_hbm.at[k, wid], idx_v)

        def chunk(j, _, k=k):
            pltpu.sync_copy(ones_v, degs[k].at[idx_v.at[j]], add=True)
            return 0

        lax.fori_loop(0, CPW, chunk, 0)
    plsc.subcore_barrier()

    for k in range(4):
        pltpu.sync_copy(
            degs[k].at[pl.ds(s * RPT, RPT)], out_hbm.at[c, k, pl.ds(s * RPT, RPT)]
        )


def _deg_call(idx_all):
    f = pl.kernel(
        _deg_body,
        out_type=jax.ShapeDtypeStruct((NC, 4, NPAD), jnp.float32),
        mesh=_mesh(),
        scratch_types=[
            pltpu.VMEM_SHARED((NPAD,), jnp.float32),
            pltpu.VMEM_SHARED((NPAD,), jnp.float32),
            pltpu.VMEM_SHARED((NPAD,), jnp.float32),
            pltpu.VMEM_SHARED((NPAD,), jnp.float32),
            pltpu.VMEM((CPW, CHUNK), jnp.int32),
            pltpu.VMEM((CHUNK,), jnp.float32),
            pltpu.VMEM((RPT,), jnp.float32),
        ],
        name="gcn_degrees_sc",
        compiler_params=pltpu.CompilerParams(use_tc_tiling_on_sc=False),
    )
    return f(idx_all)


def _msg_body(D, idx_hbm, table_hbm, out_hbm, a0, a1, tbl_sh, src_v, dst_v,
              rows_v, zero_v, sem):
    c = lax.axis_index("c")
    s = lax.axis_index("s")
    wid = c * NS + s
    aggs = [a0, a1]

    _zero_rows(zero_v, RPT, D)
    for g in range(2):
        pltpu.sync_copy(
            table_hbm.at[g, pl.ds(s * RPT, RPT)], tbl_sh.at[g, pl.ds(s * RPT, RPT)]
        )
        pltpu.sync_copy(zero_v, aggs[g].at[pl.ds(s * RPT, RPT)])
    plsc.subcore_barrier()

    for g in range(2):
        pltpu.sync_copy(idx_hbm.at[2 * g, wid], src_v)
        pltpu.sync_copy(idx_hbm.at[2 * g + 1, wid], dst_v)

        def chunk(j, _, g=g):
            pltpu.async_copy(tbl_sh.at[g].at[src_v.at[j]], rows_v, sem).wait()
            pltpu.sync_copy(rows_v, aggs[g].at[dst_v.at[j]], add=True)
            return 0

        lax.fori_loop(0, CPW, chunk, 0)
    plsc.subcore_barrier()

    for g in range(2):
        pltpu.sync_copy(
            aggs[g].at[pl.ds(s * RPT, RPT)], out_hbm.at[c, g, pl.ds(s * RPT, RPT)]
        )


def _msg_call(idx_all, table, D, tag):
    f = pl.kernel(
        functools.partial(_msg_body, D),
        out_type=jax.ShapeDtypeStruct((NC, 2, NPAD, D), jnp.float32),
        mesh=_mesh(),
        scratch_types=[
            pltpu.VMEM_SHARED((NPAD, D), jnp.float32),
            pltpu.VMEM_SHARED((NPAD, D), jnp.float32),
            pltpu.VMEM_SHARED((2, NPAD, D), jnp.float32),
            pltpu.VMEM((CPW, CHUNK), jnp.int32),
            pltpu.VMEM((CPW, CHUNK), jnp.int32),
            pltpu.VMEM((CHUNK, D), jnp.float32),
            pltpu.VMEM((RPT, D), jnp.float32),
            pltpu.SemaphoreType.DMA,
        ],
        name=f"gcn_msgpass_{tag}_sc",
        compiler_params=pltpu.CompilerParams(use_tc_tiling_on_sc=False),
    )
    return f(idx_all, table)


def _projA_body(sux, sox, w1, degp, table_out, norms_out):
    deg = degp[0] + degp[1]
    norms = jnp.where(deg > 0.0, lax.rsqrt(jnp.maximum(deg, 1e-30)), 0.0)
    norms_out[...] = norms
    h_su = jnp.dot(sux[...], w1[...], preferred_element_type=jnp.float32)
    h_so = jnp.dot(sox[...], w1[...], preferred_element_type=jnp.float32)
    table_out[0] = h_su * norms[0][:, None]
    table_out[1] = h_so * norms[2][:, None]


def _projA_call(sux_pad, sox_pad, w1, deg_partials):
    return pl.pallas_call(
        _projA_body,
        out_shape=(
            jax.ShapeDtypeStruct((2, NPAD, NHID), jnp.float32),
            jax.ShapeDtypeStruct((4, NPAD), jnp.float32),
        ),
        name="gcn_proj1_tc",
    )(sux_pad, sox_pad, w1, deg_partials)


def _projB_body(agg1, norms, b1, w2, f1_out, table2_out):
    for g in range(2):
        agg = agg1[0, g] + agg1[1, g]
        f1 = jnp.maximum(agg * norms[2 * g + 1][:, None] + b1[...][None, :], 0.0)
        f1_out[g] = f1
        h2 = jnp.dot(f1, w2[...], preferred_element_type=jnp.float32)
        table2_out[g] = h2 * norms[2 * g][:, None]


def _projB_call(agg1, norms, b1, w2):
    return pl.pallas_call(
        _projB_body,
        out_shape=(
            jax.ShapeDtypeStruct((2, NPAD, NHID), jnp.float32),
            jax.ShapeDtypeStruct((2, NPAD, NCLASS), jnp.float32),
        ),
        name="gcn_proj2_tc",
    )(agg1, norms, b1, w2)


NB = 2048


def _fin2_body(f1, agg2, norms, b2, cat_out):
    nrm = norms[...]
    b2v = b2[...]
    for g in range(2):
        f = (agg2[0, g] + agg2[1, g]) * nrm[2 * g + 1][:, None] + b2v[None, :]
        cat_out[g] = jnp.concatenate([f1[g], f], axis=1)


def _fin2_call(f1, agg2, norms, b2):
    in_specs = [
        pl.BlockSpec((2, NB, NHID), lambda k: (0, k, 0)),
        pl.BlockSpec((NC, 2, NB, NCLASS), lambda k: (0, 0, k, 0)),
        pl.BlockSpec((4, NB), lambda k: (0, k)),
        pl.BlockSpec((NCLASS,), lambda k: (0,)),
    ]
    return pl.pallas_call(
        _fin2_body,
        grid=(NPAD // NB,),
        in_specs=in_specs,
        out_specs=pl.BlockSpec((2, NB, NHID + NCLASS), lambda k: (0, k, 0)),
        out_shape=jax.ShapeDtypeStruct((2, NPAD, NHID + NCLASS), jnp.float32),
        name="gcn_finish2_tc",
    )(f1, agg2, norms, b2)


MB = 128


def _pool_body(l_su, l_so, cat, fc1w, fc1b, fc2w, fc2b, fc3w, fc3b, out):
    p_su = jnp.dot(l_su[...], cat[0], preferred_element_type=jnp.float32)
    p_so = jnp.dot(l_so[...], cat[1], preferred_element_type=jnp.float32)
    data = jnp.concatenate([p_su, p_so], axis=1)
    d1 = jnp.maximum(jnp.dot(data, fc1w[...], preferred_element_type=jnp.float32)
                     + fc1b[...][None, :], 0.0)
    d2 = jnp.maximum(jnp.dot(d1, fc2w[...], preferred_element_type=jnp.float32)
                     + fc2b[...][None, :], 0.0)
    out[...] = jnp.dot(d2, fc3w[...], preferred_element_type=jnp.float32) \
        + fc3b[...][None, :]


def _pool_call(l_su, l_so, cat, fcs):
    nblk = BATCH // MB
    in_specs = [
        pl.BlockSpec((MB, N), lambda k: (k, 0)),
        pl.BlockSpec((MB, N), lambda k: (k, 0)),
        pl.BlockSpec((2, N, NHID + NCLASS), lambda k: (0, 0, 0)),
        pl.BlockSpec((96, 64), lambda k: (0, 0)),
        pl.BlockSpec((64,), lambda k: (0,)),
        pl.BlockSpec((64, 16), lambda k: (0, 0)),
        pl.BlockSpec((16,), lambda k: (0,)),
        pl.BlockSpec((16, 1), lambda k: (0, 0)),
        pl.BlockSpec((1,), lambda k: (0,)),
    ]
    return pl.pallas_call(
        _pool_body,
        grid=(nblk,),
        in_specs=in_specs,
        out_specs=pl.BlockSpec((MB, 1), lambda k: (k, 0)),
        out_shape=jax.ShapeDtypeStruct((BATCH, 1), jnp.float32),
        name="gcn_pool_mlp_tc",
    )(l_su, l_so, cat, *fcs)


def _prep_idx(edge_index):
    pad = jnp.full((2, EPAD - E), DUMMY, jnp.int32)
    idx = jnp.concatenate([edge_index.astype(jnp.int32), pad], axis=1)
    return idx.reshape(2, NW, CPW, CHUNK)


def kernel(solute_x, solute_edge_index, solvent_x, solvent_edge_index,
           solute_len_matrix, solvent_len_matrix, W1, b1, W2, b2,
           fc1_W, fc1_b, fc2_W, fc2_b, fc3_W, fc3_b):
    idx_all = jnp.concatenate(
        [_prep_idx(solute_edge_index), _prep_idx(solvent_edge_index)], axis=0
    )
    sux = jnp.pad(solute_x, ((0, NPAD - N), (0, 0)))
    sox = jnp.pad(solvent_x, ((0, NPAD - N), (0, 0)))

    deg_partials = _deg_call(idx_all)
    table1, norms = _projA_call(sux, sox, W1, deg_partials)
    agg1 = _msg_call(idx_all, table1, NHID, "l1")
    f1, table2 = _projB_call(agg1, norms, b1, W2)
    agg2 = _msg_call(idx_all, table2, NCLASS, "l2")

    cat = _fin2_call(f1, agg2, norms, b2)
    fcs = (fc1_W, fc1_b, fc2_W, fc2_b, fc3_W, fc3_b)
    return _pool_call(solute_len_matrix, solvent_len_matrix, cat[:, :N, :], fcs)


# speedup vs baseline: 13.5035x; 1.0395x over previous
import functools

import jax
import jax.numpy as jnp
from jax import lax
from jax.experimental import pallas as pl
from jax.experimental.pallas import tpu as pltpu
from jax.experimental.pallas import tpu_sc as plsc

N = 10000
E = 320000
NFEAT = 128
NHID = 32
NCLASS = 16
BATCH = 1024

NPAD = 10240
DUMMY = N
NC = 2
NS = 16
NW = NC * NS
CHUNK = 128
CPW = 80
CPWI = 82
EPW = CPW * CHUNK
EPAD = EPW * NW
RPT = NPAD // NS
KB = 2000


def _mesh():
    return plsc.VectorSubcoreMesh(
        core_axis_name="c", subcore_axis_name="s", num_cores=NC, num_subcores=NS
    )


def _zero_rows(ref, nrows, width):
    z = jnp.zeros((16,), jnp.float32)

    def body(i, _):
        for h in range(width // 16):
            ref[i, pl.ds(16 * h, 16)] = z
        return 0

    lax.fori_loop(0, nrows, body, 0)


def _zero_flat(ref, nwords):
    z = jnp.zeros((16,), jnp.float32)

    def body(i, _):
        ref[pl.ds(16 * i, 16)] = z
        return 0

    lax.fori_loop(0, nwords // 16, body, 0)


def _deg_body(idx_hbm, out_hbm, d0, d1, d2, d3, idx_v, ones_v, zero_v,
              s0, s1, s2, s3):
    c = lax.axis_index("c")
    s = lax.axis_index("s")
    wid = c * NS + s
    degs = [d0, d1, d2, d3]
    sems = [s0, s1, s2, s3]

    for h in range(CHUNK // 16):
        ones_v[pl.ds(16 * h, 16)] = jnp.ones((16,), jnp.float32)
    _zero_flat(zero_v, RPT)
    for k in range(4):
        pltpu.sync_copy(zero_v, degs[k].at[pl.ds(s * RPT, RPT)])
    plsc.subcore_barrier()

    for k in range(4):
        pltpu.sync_copy(idx_hbm.at[k, wid], idx_v)
        for b in range(4):
            pltpu.async_copy(ones_v, degs[k].at[idx_v.at[b]], sems[b], add=True)

        def body(i, _, k=k):
            for b in range(4):
                j = 4 * i + b
                pltpu.make_async_copy(
                    ones_v, degs[k].at[idx_v.at[j - 4]], sems[b]
                ).wait()
                pltpu.async_copy(ones_v, degs[k].at[idx_v.at[j]], sems[b],
                                 add=True)
            return 0

        lax.fori_loop(1, CPW // 4, body, 0)
        for b in range(4):
            pltpu.make_async_copy(ones_v, degs[k].at[idx_v.at[b]], sems[b]).wait()
    plsc.subcore_barrier()

    for k in range(4):
        pltpu.sync_copy(
            degs[k].at[pl.ds(s * RPT, RPT)], out_hbm.at[c, k, pl.ds(s * RPT, RPT)]
        )


def _deg_call(idx_all):
    f = pl.kernel(
        _deg_body,
        out_type=jax.ShapeDtypeStruct((NC, 4, NPAD), jnp.float32),
        mesh=_mesh(),
        scratch_types=[
            pltpu.VMEM_SHARED((NPAD,), jnp.float32),
            pltpu.VMEM_SHARED((NPAD,), jnp.float32),
            pltpu.VMEM_SHARED((NPAD,), jnp.float32),
            pltpu.VMEM_SHARED((NPAD,), jnp.float32),
            pltpu.VMEM((CPWI, CHUNK), jnp.int32),
            pltpu.VMEM((CHUNK,), jnp.float32),
            pltpu.VMEM((RPT,), jnp.float32),
            pltpu.SemaphoreType.DMA,
            pltpu.SemaphoreType.DMA,
            pltpu.SemaphoreType.DMA,
            pltpu.SemaphoreType.DMA,
        ],
        name="gcn_degrees_sc",
        compiler_params=pltpu.CompilerParams(use_tc_tiling_on_sc=False),
    )
    return f(idx_all)


def _msg_body(D, idx_hbm, table_hbm, out_hbm, agg, tbl_sh, src_v, dst_v,
              r0, r1, zero_v, gs0, gs1):
    c = lax.axis_index("c")
    s = lax.axis_index("s")
    wid = c * NS + s

    _zero_rows(zero_v, RPT, D)
    for g in range(2):
        pltpu.sync_copy(
            table_hbm.at[g, pl.ds(s * RPT, RPT)], tbl_sh.at[pl.ds(s * RPT, RPT)]
        )
        pltpu.sync_copy(zero_v, agg.at[pl.ds(s * RPT, RPT)])
        pltpu.sync_copy(idx_hbm.at[2 * g, wid], src_v)
        pltpu.sync_copy(idx_hbm.at[2 * g + 1, wid], dst_v)
        plsc.subcore_barrier()

        hdum = table_hbm.at[g, pl.ds(0, CHUNK)]
        pltpu.async_copy(tbl_sh.at[src_v.at[0]], r0, gs0)
        pltpu.async_copy(tbl_sh.at[src_v.at[1]], r1, gs1)

        def body(i, _):
            j0 = 2 * i
            pltpu.make_async_copy(hdum, r0, gs0).wait()
            pltpu.sync_copy(r0, agg.at[dst_v.at[j0]], add=True)
            pltpu.async_copy(tbl_sh.at[src_v.at[j0 + 2]], r0, gs0)
            pltpu.make_async_copy(hdum, r1, gs1).wait()
            pltpu.sync_copy(r1, agg.at[dst_v.at[j0 + 1]], add=True)
            pltpu.async_copy(tbl_sh.at[src_v.at[j0 + 3]], r1, gs1)
            return 0

        lax.fori_loop(0, CPW // 2, body, 0)
        pltpu.make_async_copy(hdum, r0, gs0).wait()
        pltpu.make_async_copy(hdum, r1, gs1).wait()
        plsc.subcore_barrier()

        pltpu.sync_copy(
            agg.at[pl.ds(s * RPT, RPT)], out_hbm.at[c, g, pl.ds(s * RPT, RPT)]
        )


def _msg_call(idx_all, table, D, tag):
    f = pl.kernel(
        functools.partial(_msg_body, D),
        out_type=jax.ShapeDtypeStruct((NC, 2, NPAD, D), jnp.float32),
        mesh=_mesh(),
        scratch_types=[
            pltpu.VMEM_SHARED((NPAD, D), jnp.float32),
            pltpu.VMEM_SHARED((NPAD, D), jnp.float32),
            pltpu.VMEM((CPWI, CHUNK), jnp.int32),
            pltpu.VMEM((CPWI, CHUNK), jnp.int32),
            pltpu.VMEM((CHUNK, D), jnp.float32),
            pltpu.VMEM((CHUNK, D), jnp.float32),
            pltpu.VMEM((RPT, D), jnp.float32),
            pltpu.SemaphoreType.DMA,
            pltpu.SemaphoreType.DMA,
        ],
        name=f"gcn_msgpass_{tag}_sc",
        compiler_params=pltpu.CompilerParams(use_tc_tiling_on_sc=False),
    )
    return f(idx_all, table)


def _projA_body(sux, sox, w1, degp, table_out, norms_out):
    deg = degp[0] + degp[1]
    norms = jnp.where(deg > 0.0, lax.rsqrt(jnp.maximum(deg, 1e-30)), 0.0)
    norms_out[...] = norms
    h_su = jnp.dot(sux[...], w1[...], preferred_element_type=jnp.float32)
    h_so = jnp.dot(sox[...], w1[...], preferred_element_type=jnp.float32)
    table_out[0] = h_su * norms[0][:, None]
    table_out[1] = h_so * norms[2][:, None]


def _projA_call(sux_pad, sox_pad, w1, deg_partials):
    return pl.pallas_call(
        _projA_body,
        out_shape=(
            jax.ShapeDtypeStruct((2, NPAD, NHID), jnp.float32),
            jax.ShapeDtypeStruct((4, NPAD), jnp.float32),
        ),
        name="gcn_proj1_tc",
    )(sux_pad, sox_pad, w1, deg_partials)


def _projB_body(agg1, norms, b1, w2, f1_out, table2_out):
    for g in range(2):
        agg = agg1[0, g] + agg1[1, g]
        f1 = jnp.maximum(agg * norms[2 * g + 1][:, None] + b1[...][None, :], 0.0)
        f1_out[g] = f1
        h2 = jnp.dot(f1, w2[...], preferred_element_type=jnp.float32)
        table2_out[g] = h2 * norms[2 * g][:, None]


def _projB_call(agg1, norms, b1, w2):
    return pl.pallas_call(
        _projB_body,
        out_shape=(
            jax.ShapeDtypeStruct((2, NPAD, NHID), jnp.float32),
            jax.ShapeDtypeStruct((2, NPAD, NCLASS), jnp.float32),
        ),
        name="gcn_proj2_tc",
    )(agg1, norms, b1, w2)


NB = 2048


def _fin2_body(f1, agg2, norms, b2, cat_out):
    nrm = norms[...]
    b2v = b2[...]
    for g in range(2):
        f = (agg2[0, g] + agg2[1, g]) * nrm[2 * g + 1][:, None] + b2v[None, :]
        cat_out[g] = jnp.concatenate([f1[g], f], axis=1)


def _fin2_call(f1, agg2, norms, b2):
    in_specs = [
        pl.BlockSpec((2, NB, NHID), lambda k: (0, k, 0)),
        pl.BlockSpec((NC, 2, NB, NCLASS), lambda k: (0, 0, k, 0)),
        pl.BlockSpec((4, NB), lambda k: (0, k)),
        pl.BlockSpec((NCLASS,), lambda k: (0,)),
    ]
    return pl.pallas_call(
        _fin2_body,
        grid=(NPAD // NB,),
        in_specs=in_specs,
        out_specs=pl.BlockSpec((2, NB, NHID + NCLASS), lambda k: (0, k, 0)),
        out_shape=jax.ShapeDtypeStruct((2, NPAD, NHID + NCLASS), jnp.float32),
        name="gcn_finish2_tc",
    )(f1, agg2, norms, b2)


MB = 128


def _pool_body(l_su, l_so, cat, fc1w, fc1b, fc2w, fc2b, fc3w, fc3b, out):
    p_su = jnp.dot(l_su[...], cat[0], preferred_element_type=jnp.float32)
    p_so = jnp.dot(l_so[...], cat[1], preferred_element_type=jnp.float32)
    data = jnp.concatenate([p_su, p_so], axis=1)
    d1 = jnp.maximum(jnp.dot(data, fc1w[...], preferred_element_type=jnp.float32)
                     + fc1b[...][None, :], 0.0)
    d2 = jnp.maximum(jnp.dot(d1, fc2w[...], preferred_element_type=jnp.float32)
                     + fc2b[...][None, :], 0.0)
    out[...] = jnp.dot(d2, fc3w[...], preferred_element_type=jnp.float32) \
        + fc3b[...][None, :]


def _pool_call(l_su, l_so, cat, fcs):
    nblk = BATCH // MB
    in_specs = [
        pl.BlockSpec((MB, N), lambda k: (k, 0)),
        pl.BlockSpec((MB, N), lambda k: (k, 0)),
        pl.BlockSpec((2, N, NHID + NCLASS), lambda k: (0, 0, 0)),
        pl.BlockSpec((96, 64), lambda k: (0, 0)),
        pl.BlockSpec((64,), lambda k: (0,)),
        pl.BlockSpec((64, 16), lambda k: (0, 0)),
        pl.BlockSpec((16,), lambda k: (0,)),
        pl.BlockSpec((16, 1), lambda k: (0, 0)),
        pl.BlockSpec((1,), lambda k: (0,)),
    ]
    return pl.pallas_call(
        _pool_body,
        grid=(nblk,),
        in_specs=in_specs,
        out_specs=pl.BlockSpec((MB, 1), lambda k: (k, 0)),
        out_shape=jax.ShapeDtypeStruct((BATCH, 1), jnp.float32),
        name="gcn_pool_mlp_tc",
    )(l_su, l_so, cat, *fcs)


def _prep_idx(edge_index):
    pad = jnp.full((2, EPAD - E), DUMMY, jnp.int32)
    idx = jnp.concatenate([edge_index.astype(jnp.int32), pad], axis=1)
    idx = idx.reshape(2, NW, CPW, CHUNK)
    look = jnp.full((2, NW, CPWI - CPW, CHUNK), DUMMY, jnp.int32)
    return jnp.concatenate([idx, look], axis=2)


def kernel(solute_x, solute_edge_index, solvent_x, solvent_edge_index,
           solute_len_matrix, solvent_len_matrix, W1, b1, W2, b2,
           fc1_W, fc1_b, fc2_W, fc2_b, fc3_W, fc3_b):
    idx_all = jnp.concatenate(
        [_prep_idx(solute_edge_index), _prep_idx(solvent_edge_index)], axis=0
    )
    sux = jnp.pad(solute_x, ((0, NPAD - N), (0, 0)))
    sox = jnp.pad(solvent_x, ((0, NPAD - N), (0, 0)))

    deg_partials = _deg_call(idx_all)
    table1, norms = _projA_call(sux, sox, W1, deg_partials)
    agg1 = _msg_call(idx_all, table1, NHID, "l1")
    f1, table2 = _projB_call(agg1, norms, b1, W2)
    agg2 = _msg_call(idx_all, table2, NCLASS, "l2")

    cat = _fin2_call(f1, agg2, norms, b2)
    fcs = (fc1_W, fc1_b, fc2_W, fc2_b, fc3_W, fc3_b)
    return _pool_call(solute_len_matrix, solvent_len_matrix, cat[:, :N, :], fcs)


# speedup vs baseline: 14.0723x; 1.0421x over previous
import functools

import jax
import jax.numpy as jnp
from jax import lax
from jax.experimental import pallas as pl
from jax.experimental.pallas import tpu as pltpu
from jax.experimental.pallas import tpu_sc as plsc

N = 10000
E = 320000
NFEAT = 128
NHID = 32
NCLASS = 16
BATCH = 1024

NPAD = 10240
DUMMY = N
NC = 2
NS = 16
NW = NC * NS
CHUNK = 128
CPW = 80
CPWI = 82
EPW = CPW * CHUNK
EPAD = EPW * NW
RPT = NPAD // NS
KB = 2000


def _mesh():
    return plsc.VectorSubcoreMesh(
        core_axis_name="c", subcore_axis_name="s", num_cores=NC, num_subcores=NS
    )


def _zero_rows(ref, nrows, width):
    z = jnp.zeros((16,), jnp.float32)

    def body(i, _):
        for h in range(width // 16):
            ref[i, pl.ds(16 * h, 16)] = z
        return 0

    lax.fori_loop(0, nrows, body, 0)


def _zero_flat(ref, nwords):
    z = jnp.zeros((16,), jnp.float32)

    def body(i, _):
        ref[pl.ds(16 * i, 16)] = z
        return 0

    lax.fori_loop(0, nwords // 16, body, 0)


def _deg_body(idx_hbm, out_hbm, d0, d1, d2, d3, idx_v, ones_v, zero_v,
              s0, s1, s2, s3):
    c = lax.axis_index("c")
    s = lax.axis_index("s")
    wid = c * NS + s
    degs = [d0, d1, d2, d3]
    sems = [s0, s1, s2, s3]

    for h in range(CHUNK // 16):
        ones_v[pl.ds(16 * h, 16)] = jnp.ones((16,), jnp.float32)
    _zero_flat(zero_v, RPT)
    for k in range(4):
        pltpu.sync_copy(zero_v, degs[k].at[pl.ds(s * RPT, RPT)])
    plsc.subcore_barrier()

    for k in range(4):
        pltpu.sync_copy(idx_hbm.at[k, wid], idx_v)
        for b in range(4):
            pltpu.async_copy(ones_v, degs[k].at[idx_v.at[b]], sems[b], add=True)

        def body(i, _, k=k):
            for b in range(4):
                j = 4 * i + b
                pltpu.make_async_copy(
                    ones_v, degs[k].at[idx_v.at[j - 4]], sems[b]
                ).wait()
                pltpu.async_copy(ones_v, degs[k].at[idx_v.at[j]], sems[b],
                                 add=True)
            return 0

        lax.fori_loop(1, CPW // 4, body, 0)
        for b in range(4):
            pltpu.make_async_copy(ones_v, degs[k].at[idx_v.at[b]], sems[b]).wait()
    plsc.subcore_barrier()

    for k in range(4):
        pltpu.sync_copy(
            degs[k].at[pl.ds(s * RPT, RPT)], out_hbm.at[c, k, pl.ds(s * RPT, RPT)]
        )


def _deg_call(idx_all):
    f = pl.kernel(
        _deg_body,
        out_type=jax.ShapeDtypeStruct((NC, 4, NPAD), jnp.float32),
        mesh=_mesh(),
        scratch_types=[
            pltpu.VMEM_SHARED((NPAD,), jnp.float32),
            pltpu.VMEM_SHARED((NPAD,), jnp.float32),
            pltpu.VMEM_SHARED((NPAD,), jnp.float32),
            pltpu.VMEM_SHARED((NPAD,), jnp.float32),
            pltpu.VMEM((CPWI, CHUNK), jnp.int32),
            pltpu.VMEM((CHUNK,), jnp.float32),
            pltpu.VMEM((RPT,), jnp.float32),
            pltpu.SemaphoreType.DMA,
            pltpu.SemaphoreType.DMA,
            pltpu.SemaphoreType.DMA,
            pltpu.SemaphoreType.DMA,
        ],
        name="gcn_degrees_sc",
        compiler_params=pltpu.CompilerParams(use_tc_tiling_on_sc=False),
    )
    return f(idx_all)


def _msg_body(D, idx_hbm, table_hbm, out_hbm, agg, tbl_sh, src_v, dst_v,
              r0, r1, zero_v, gs0, gs1):
    c = lax.axis_index("c")
    s = lax.axis_index("s")
    wid = c * NS + s

    _zero_rows(zero_v, RPT, D)
    for g in range(2):
        pltpu.sync_copy(
            table_hbm.at[g, pl.ds(s * RPT, RPT)], tbl_sh.at[pl.ds(s * RPT, RPT)]
        )
        pltpu.sync_copy(zero_v, agg.at[pl.ds(s * RPT, RPT)])
        pltpu.sync_copy(idx_hbm.at[2 * g, wid], src_v)
        pltpu.sync_copy(idx_hbm.at[2 * g + 1, wid], dst_v)
        plsc.subcore_barrier()

        hdum = table_hbm.at[g, pl.ds(0, CHUNK)]
        pltpu.async_copy(tbl_sh.at[src_v.at[0]], r0, gs0)
        pltpu.async_copy(tbl_sh.at[src_v.at[1]], r1, gs1)

        def body(i, _):
            j0 = 2 * i
            pltpu.make_async_copy(hdum, r0, gs0).wait()
            pltpu.sync_copy(r0, agg.at[dst_v.at[j0]], add=True)
            pltpu.async_copy(tbl_sh.at[src_v.at[j0 + 2]], r0, gs0)
            pltpu.make_async_copy(hdum, r1, gs1).wait()
            pltpu.sync_copy(r1, agg.at[dst_v.at[j0 + 1]], add=True)
            pltpu.async_copy(tbl_sh.at[src_v.at[j0 + 3]], r1, gs1)
            return 0

        lax.fori_loop(0, CPW // 2, body, 0)
        pltpu.make_async_copy(hdum, r0, gs0).wait()
        pltpu.make_async_copy(hdum, r1, gs1).wait()
        plsc.subcore_barrier()

        pltpu.sync_copy(
            agg.at[pl.ds(s * RPT, RPT)], out_hbm.at[c, g, pl.ds(s * RPT, RPT)]
        )


def _msg_call(idx_all, table, D, tag):
    f = pl.kernel(
        functools.partial(_msg_body, D),
        out_type=jax.ShapeDtypeStruct((NC, 2, NPAD, D), jnp.float32),
        mesh=_mesh(),
        scratch_types=[
            pltpu.VMEM_SHARED((NPAD, D), jnp.float32),
            pltpu.VMEM_SHARED((NPAD, D), jnp.float32),
            pltpu.VMEM((CPWI, CHUNK), jnp.int32),
            pltpu.VMEM((CPWI, CHUNK), jnp.int32),
            pltpu.VMEM((CHUNK, D), jnp.float32),
            pltpu.VMEM((CHUNK, D), jnp.float32),
            pltpu.VMEM((RPT, D), jnp.float32),
            pltpu.SemaphoreType.DMA,
            pltpu.SemaphoreType.DMA,
        ],
        name=f"gcn_msgpass_{tag}_sc",
        compiler_params=pltpu.CompilerParams(use_tc_tiling_on_sc=False),
    )
    return f(idx_all, table)


def _projA_body(sux, sox, w1, degp, table_out, norms_out):
    deg = degp[0] + degp[1]
    norms = jnp.where(deg > 0.0, lax.rsqrt(jnp.maximum(deg, 1e-30)), 0.0)
    norms_out[...] = norms
    h_su = jnp.dot(sux[...], w1[...], preferred_element_type=jnp.float32)
    h_so = jnp.dot(sox[...], w1[...], preferred_element_type=jnp.float32)
    table_out[0] = h_su * norms[0][:, None]
    table_out[1] = h_so * norms[2][:, None]


def _projA_call(sux_pad, sox_pad, w1, deg_partials):
    return pl.pallas_call(
        _projA_body,
        out_shape=(
            jax.ShapeDtypeStruct((2, NPAD, NHID), jnp.float32),
            jax.ShapeDtypeStruct((4, NPAD), jnp.float32),
        ),
        name="gcn_proj1_tc",
    )(sux_pad, sox_pad, w1, deg_partials)


def _projB_body(agg1, norms, b1, w2, f1_out, table2_out):
    for g in range(2):
        agg = agg1[0, g] + agg1[1, g]
        f1 = jnp.maximum(agg * norms[2 * g + 1][:, None] + b1[...][None, :], 0.0)
        f1_out[g] = f1
        h2 = jnp.dot(f1, w2[...], preferred_element_type=jnp.float32)
        table2_out[g] = h2 * norms[2 * g][:, None]


def _projB_call(agg1, norms, b1, w2):
    return pl.pallas_call(
        _projB_body,
        out_shape=(
            jax.ShapeDtypeStruct((2, NPAD, NHID), jnp.float32),
            jax.ShapeDtypeStruct((2, NPAD, NCLASS), jnp.float32),
        ),
        name="gcn_proj2_tc",
    )(agg1, norms, b1, w2)


NB = 2048


def _fin2_body(f1, agg2, norms, b2, cat_out):
    nrm = norms[...]
    b2v = b2[...]
    for g in range(2):
        f = (agg2[0, g] + agg2[1, g]) * nrm[2 * g + 1][:, None] + b2v[None, :]
        cat_out[g] = jnp.concatenate([f1[g], f], axis=1)


def _fin2_call(f1, agg2, norms, b2):
    in_specs = [
        pl.BlockSpec((2, NB, NHID), lambda k: (0, k, 0)),
        pl.BlockSpec((NC, 2, NB, NCLASS), lambda k: (0, 0, k, 0)),
        pl.BlockSpec((4, NB), lambda k: (0, k)),
        pl.BlockSpec((NCLASS,), lambda k: (0,)),
    ]
    return pl.pallas_call(
        _fin2_body,
        grid=(NPAD // NB,),
        in_specs=in_specs,
        out_specs=pl.BlockSpec((2, NB, NHID + NCLASS), lambda k: (0, k, 0)),
        out_shape=jax.ShapeDtypeStruct((2, NPAD, NHID + NCLASS), jnp.float32),
        name="gcn_finish2_tc",
    )(f1, agg2, norms, b2)


MB = 128


def _pool_body(l_suT, l_soT, cat, fc1w, fc1b, fc2w, fc2b, fc3w, fc3b, out):
    dn = (((0,), (0,)), ((), ()))
    p_su = lax.dot_general(l_suT[...], cat[0], dn,
                           preferred_element_type=jnp.float32)
    p_so = lax.dot_general(l_soT[...], cat[1], dn,
                           preferred_element_type=jnp.float32)
    data = jnp.concatenate([p_su, p_so], axis=1)
    d1 = jnp.maximum(jnp.dot(data, fc1w[...], preferred_element_type=jnp.float32)
                     + fc1b[...][None, :], 0.0)
    d2 = jnp.maximum(jnp.dot(d1, fc2w[...], preferred_element_type=jnp.float32)
                     + fc2b[...][None, :], 0.0)
    out[...] = jnp.dot(d2, fc3w[...], preferred_element_type=jnp.float32) \
        + fc3b[...][None, :]


def _pool_call(l_suT, l_soT, cat, fcs):
    nblk = BATCH // MB
    in_specs = [
        pl.BlockSpec((N, MB), lambda k: (0, k)),
        pl.BlockSpec((N, MB), lambda k: (0, k)),
        pl.BlockSpec((2, N, NHID + NCLASS), lambda k: (0, 0, 0)),
        pl.BlockSpec((96, 64), lambda k: (0, 0)),
        pl.BlockSpec((64,), lambda k: (0,)),
        pl.BlockSpec((64, 16), lambda k: (0, 0)),
        pl.BlockSpec((16,), lambda k: (0,)),
        pl.BlockSpec((16, 1), lambda k: (0, 0)),
        pl.BlockSpec((1,), lambda k: (0,)),
    ]
    return pl.pallas_call(
        _pool_body,
        grid=(nblk,),
        in_specs=in_specs,
        out_specs=pl.BlockSpec((MB, 1), lambda k: (k, 0)),
        out_shape=jax.ShapeDtypeStruct((BATCH, 1), jnp.float32),
        name="gcn_pool_mlp_tc",
    )(l_suT, l_soT, cat, *fcs)


def _prep_idx(edge_index):
    pad = jnp.full((2, EPAD - E), DUMMY, jnp.int32)
    idx = jnp.concatenate([edge_index.astype(jnp.int32), pad], axis=1)
    idx = idx.reshape(2, NW, CPW, CHUNK)
    look = jnp.full((2, NW, CPWI - CPW, CHUNK), DUMMY, jnp.int32)
    return jnp.concatenate([idx, look], axis=2)


def kernel(solute_x, solute_edge_index, solvent_x, solvent_edge_index,
           solute_len_matrix, solvent_len_matrix, W1, b1, W2, b2,
           fc1_W, fc1_b, fc2_W, fc2_b, fc3_W, fc3_b):
    idx_all = jnp.concatenate(
        [_prep_idx(solute_edge_index), _prep_idx(solvent_edge_index)], axis=0
    )
    sux = jnp.pad(solute_x, ((0, NPAD - N), (0, 0)))
    sox = jnp.pad(solvent_x, ((0, NPAD - N), (0, 0)))

    deg_partials = _deg_call(idx_all)
    table1, norms = _projA_call(sux, sox, W1, deg_partials)
    agg1 = _msg_call(idx_all, table1, NHID, "l1")
    f1, table2 = _projB_call(agg1, norms, b1, W2)
    agg2 = _msg_call(idx_all, table2, NCLASS, "l2")

    cat = _fin2_call(f1, agg2, norms, b2)
    fcs = (fc1_W, fc1_b, fc2_W, fc2_b, fc3_W, fc3_b)
    return _pool_call(solute_len_matrix.T, solvent_len_matrix.T,
                      cat[:, :N, :], fcs)


# speedup vs baseline: 14.9981x; 1.0658x over previous
import functools

import jax
import jax.numpy as jnp
from jax import lax
from jax.experimental import pallas as pl
from jax.experimental.pallas import tpu as pltpu
from jax.experimental.pallas import tpu_sc as plsc

N = 10000
E = 320000
NFEAT = 128
NHID = 32
NCLASS = 16
BATCH = 1024

NPAD = 10240
DUMMY = N
NC = 2
NS = 16
NW = NC * NS
CHUNK = 128
CPW = 80
CPWI = 82
EPW = CPW * CHUNK
EPAD = EPW * NW
RPT = NPAD // NS


def _mesh():
    return plsc.VectorSubcoreMesh(
        core_axis_name="c", subcore_axis_name="s", num_cores=NC, num_subcores=NS
    )


def _zero_rows(ref, nrows, width):
    z = jnp.zeros((16,), jnp.float32)

    def body(i, _):
        for h in range(width // 16):
            ref[i, pl.ds(16 * h, 16)] = z
        return 0

    lax.fori_loop(0, nrows, body, 0)


def _zero_flat(ref, nwords):
    z = jnp.zeros((16,), jnp.float32)

    def body(i, _):
        ref[pl.ds(16 * i, 16)] = z
        return 0

    lax.fori_loop(0, nwords // 16, body, 0)


def _deg_body(idx_hbm, out_hbm, d0, d1, d2, d3, idx_v, ones_v, zero_v,
              s0, s1, s2, s3):
    c = lax.axis_index("c")
    s = lax.axis_index("s")
    wid = c * NS + s
    degs = [d0, d1, d2, d3]
    sems = [s0, s1, s2, s3]

    for h in range(CHUNK // 16):
        ones_v[pl.ds(16 * h, 16)] = jnp.ones((16,), jnp.float32)
    _zero_flat(zero_v, RPT)
    for k in range(4):
        pltpu.sync_copy(zero_v, degs[k].at[pl.ds(s * RPT, RPT)])
    plsc.subcore_barrier()

    for k in range(4):
        pltpu.sync_copy(idx_hbm.at[k, wid], idx_v)
        for b in range(4):
            pltpu.async_copy(ones_v, degs[k].at[idx_v.at[b]], sems[b], add=True)

        def body(i, _, k=k):
            for b in range(4):
                j = 4 * i + b
                pltpu.make_async_copy(
                    ones_v, degs[k].at[idx_v.at[j - 4]], sems[b]
                ).wait()
                pltpu.async_copy(ones_v, degs[k].at[idx_v.at[j]], sems[b],
                                 add=True)
            return 0

        lax.fori_loop(1, CPW // 4, body, 0)
        for b in range(4):
            pltpu.make_async_copy(ones_v, degs[k].at[idx_v.at[b]], sems[b]).wait()
    plsc.subcore_barrier()

    for k in range(4):
        pltpu.sync_copy(
            degs[k].at[pl.ds(s * RPT, RPT)], out_hbm.at[c, k, pl.ds(s * RPT, RPT)]
        )


def _deg_call(idx_all):
    f = pl.kernel(
        _deg_body,
        out_type=jax.ShapeDtypeStruct((NC, 4, NPAD), jnp.float32),
        mesh=_mesh(),
        scratch_types=[
            pltpu.VMEM_SHARED((NPAD,), jnp.float32),
            pltpu.VMEM_SHARED((NPAD,), jnp.float32),
            pltpu.VMEM_SHARED((NPAD,), jnp.float32),
            pltpu.VMEM_SHARED((NPAD,), jnp.float32),
            pltpu.VMEM((CPWI, CHUNK), jnp.int32),
            pltpu.VMEM((CHUNK,), jnp.float32),
            pltpu.VMEM((RPT,), jnp.float32),
            pltpu.SemaphoreType.DMA,
            pltpu.SemaphoreType.DMA,
            pltpu.SemaphoreType.DMA,
            pltpu.SemaphoreType.DMA,
        ],
        name="gcn_degrees_sc",
        compiler_params=pltpu.CompilerParams(use_tc_tiling_on_sc=False),
    )
    return f(idx_all)


def _msg_body(D, g, idx_hbm, table_hbm, out_hbm, agg, tbl_sh, src_v, dst_v,
              r0, r1, zero_v, gs0, gs1):
    c = lax.axis_index("c")
    s = lax.axis_index("s")
    wid = c * NS + s

    _zero_rows(zero_v, RPT, D)
    pltpu.sync_copy(
        table_hbm.at[g, pl.ds(s * RPT, RPT)], tbl_sh.at[pl.ds(s * RPT, RPT)]
    )
    pltpu.sync_copy(zero_v, agg.at[pl.ds(s * RPT, RPT)])
    pltpu.sync_copy(idx_hbm.at[2 * g, wid], src_v)
    pltpu.sync_copy(idx_hbm.at[2 * g + 1, wid], dst_v)
    plsc.subcore_barrier()

    hdum = table_hbm.at[g, pl.ds(0, CHUNK)]
    pltpu.async_copy(tbl_sh.at[src_v.at[0]], r0, gs0)
    pltpu.async_copy(tbl_sh.at[src_v.at[1]], r1, gs1)

    def body(i, _):
        j0 = 2 * i
        pltpu.make_async_copy(hdum, r0, gs0).wait()
        pltpu.sync_copy(r0, agg.at[dst_v.at[j0]], add=True)
        pltpu.async_copy(tbl_sh.at[src_v.at[j0 + 2]], r0, gs0)
        pltpu.make_async_copy(hdum, r1, gs1).wait()
        pltpu.sync_copy(r1, agg.at[dst_v.at[j0 + 1]], add=True)
        pltpu.async_copy(tbl_sh.at[src_v.at[j0 + 3]], r1, gs1)
        return 0

    lax.fori_loop(0, CPW // 2, body, 0)
    pltpu.make_async_copy(hdum, r0, gs0).wait()
    pltpu.make_async_copy(hdum, r1, gs1).wait()
    plsc.subcore_barrier()

    pltpu.sync_copy(
        agg.at[pl.ds(s * RPT, RPT)], out_hbm.at[c, pl.ds(s * RPT, RPT)]
    )


def _msg_call(idx_all, table, D, g, tag):
    f = pl.kernel(
        functools.partial(_msg_body, D, g),
        out_type=jax.ShapeDtypeStruct((NC, NPAD, D), jnp.float32),
        mesh=_mesh(),
        scratch_types=[
            pltpu.VMEM_SHARED((NPAD, D), jnp.float32),
            pltpu.VMEM_SHARED((NPAD, D), jnp.float32),
            pltpu.VMEM((CPWI, CHUNK), jnp.int32),
            pltpu.VMEM((CPWI, CHUNK), jnp.int32),
            pltpu.VMEM((CHUNK, D), jnp.float32),
            pltpu.VMEM((CHUNK, D), jnp.float32),
            pltpu.VMEM((RPT, D), jnp.float32),
            pltpu.SemaphoreType.DMA,
            pltpu.SemaphoreType.DMA,
        ],
        name=f"gcn_msgpass_{tag}_sc",
        compiler_params=pltpu.CompilerParams(use_tc_tiling_on_sc=False),
    )
    return f(idx_all, table)


def _h1_body(sux, sox, w1, h1_out):
    h1_out[0] = jnp.dot(sux[...], w1[...], preferred_element_type=jnp.float32)
    h1_out[1] = jnp.dot(sox[...], w1[...], preferred_element_type=jnp.float32)


def _h1_call(sux_pad, sox_pad, w1):
    return pl.pallas_call(
        _h1_body,
        out_shape=jax.ShapeDtypeStruct((2, NPAD, NHID), jnp.float32),
        name="gcn_h1_tc",
    )(sux_pad, sox_pad, w1)


def _scale1_body(h1, degp, table_out, norms_out):
    deg = degp[0] + degp[1]
    norms = jnp.where(deg > 0.0, lax.rsqrt(jnp.maximum(deg, 1e-30)), 0.0)
    norms_out[...] = norms
    table_out[0] = h1[0] * norms[0][:, None]
    table_out[1] = h1[1] * norms[2][:, None]


def _scale1_call(h1, deg_partials):
    return pl.pallas_call(
        _scale1_body,
        out_shape=(
            jax.ShapeDtypeStruct((2, NPAD, NHID), jnp.float32),
            jax.ShapeDtypeStruct((4, NPAD), jnp.float32),
        ),
        name="gcn_scale1_tc",
    )(h1, deg_partials)


def _projB_body(g, agg1, norms, b1, w2, f1_out, table2_out):
    agg = agg1[0] + agg1[1]
    f1 = jnp.maximum(agg * norms[2 * g + 1][:, None] + b1[...][None, :], 0.0)
    f1_out[...] = f1
    h2 = jnp.dot(f1, w2[...], preferred_element_type=jnp.float32)
    table2_out[g] = h2 * norms[2 * g][:, None]
    table2_out[1 - g] = jnp.zeros((NPAD, NCLASS), jnp.float32)


def _projB_call(agg1, norms, b1, w2, g, tag):
    return pl.pallas_call(
        functools.partial(_projB_body, g),
        out_shape=(
            jax.ShapeDtypeStruct((NPAD, NHID), jnp.float32),
            jax.ShapeDtypeStruct((2, NPAD, NCLASS), jnp.float32),
        ),
        name=f"gcn_proj2_{tag}_tc",
    )(agg1, norms, b1, w2)


NB = 2048


def _fin2_body(g, f1, agg2, norms, b2, cat_out):
    f = (agg2[0] + agg2[1]) * norms[2 * g + 1][:, None] + b2[...][None, :]
    cat_out[...] = jnp.concatenate([f1[...], f], axis=1)


def _fin2_call(f1_g, agg2_g, norms, b2, g, tag):
    in_specs = [
        pl.BlockSpec((NB, NHID), lambda k: (k, 0)),
        pl.BlockSpec((NC, NB, NCLASS), lambda k: (0, k, 0)),
        pl.BlockSpec((4, NB), lambda k: (0, k)),
        pl.BlockSpec((NCLASS,), lambda k: (0,)),
    ]
    return pl.pallas_call(
        functools.partial(_fin2_body, g),
        grid=(NPAD // NB,),
        in_specs=in_specs,
        out_specs=pl.BlockSpec((NB, NHID + NCLASS), lambda k: (k, 0)),
        out_shape=jax.ShapeDtypeStruct((NPAD, NHID + NCLASS), jnp.float32),
        name=f"gcn_finish2_{tag}_tc",
    )(f1_g, agg2_g, norms, b2)


MB = 128


def _pool_body(lT, cat, p_out):
    dn = (((0,), (0,)), ((), ()))
    p_out[...] = lax.dot_general(lT[...], cat[...], dn,
                                 preferred_element_type=jnp.float32)


def _pool_call(lT, cat_g, tag):
    in_specs = [
        pl.BlockSpec((N, MB), lambda k: (0, k)),
        pl.BlockSpec((N, NHID + NCLASS), lambda k: (0, 0)),
    ]
    return pl.pallas_call(
        _pool_body,
        grid=(BATCH // MB,),
        in_specs=in_specs,
        out_specs=pl.BlockSpec((MB, NHID + NCLASS), lambda k: (k, 0)),
        out_shape=jax.ShapeDtypeStruct((BATCH, NHID + NCLASS), jnp.float32),
        name=f"gcn_pool_{tag}_tc",
    )(lT, cat_g)


def _mlp_body(p_su, p_so, fc1w, fc1b, fc2w, fc2b, fc3w, fc3b, out):
    data = jnp.concatenate([p_su[...], p_so[...]], axis=1)
    d1 = jnp.maximum(jnp.dot(data, fc1w[...], preferred_element_type=jnp.float32)
                     + fc1b[...][None, :], 0.0)
    d2 = jnp.maximum(jnp.dot(d1, fc2w[...], preferred_element_type=jnp.float32)
                     + fc2b[...][None, :], 0.0)
    out[...] = jnp.dot(d2, fc3w[...], preferred_element_type=jnp.float32) \
        + fc3b[...][None, :]


def _mlp_call(p_su, p_so, fcs):
    return pl.pallas_call(
        _mlp_body,
        out_shape=jax.ShapeDtypeStruct((BATCH, 1), jnp.float32),
        name="gcn_mlp_tc",
    )(p_su, p_so, *fcs)


def _prep_idx(edge_index):
    pad = jnp.full((2, EPAD - E), DUMMY, jnp.int32)
    idx = jnp.concatenate([edge_index.astype(jnp.int32), pad], axis=1)
    idx = idx.reshape(2, NW, CPW, CHUNK)
    look = jnp.full((2, NW, CPWI - CPW, CHUNK), DUMMY, jnp.int32)
    return jnp.concatenate([idx, look], axis=2)


def kernel(solute_x, solute_edge_index, solvent_x, solvent_edge_index,
           solute_len_matrix, solvent_len_matrix, W1, b1, W2, b2,
           fc1_W, fc1_b, fc2_W, fc2_b, fc3_W, fc3_b):
    idx_all = jnp.concatenate(
        [_prep_idx(solute_edge_index), _prep_idx(solvent_edge_index)], axis=0
    )
    sux = jnp.pad(solute_x, ((0, NPAD - N), (0, 0)))
    sox = jnp.pad(solvent_x, ((0, NPAD - N), (0, 0)))

    deg_partials = _deg_call(idx_all)
    h1 = _h1_call(sux, sox, W1)
    table1, norms = _scale1_call(h1, deg_partials)

    agg1_su = _msg_call(idx_all, table1, NHID, 0, "l1su")
    f1_su, table2_su = _projB_call(agg1_su, norms, b1, W2, 0, "su")
    agg1_so = _msg_call(idx_all, table1, NHID, 1, "l1so")
    f1_so, table2_so = _projB_call(agg1_so, norms, b1, W2, 1, "so")

    agg2_su = _msg_call(idx_all, table2_su, NCLASS, 0, "l2su")
    agg2_so = _msg_call(idx_all, table2_so, NCLASS, 1, "l2so")

    cat_su = _fin2_call(f1_su, agg2_su, norms, b2, 0, "su")
    cat_so = _fin2_call(f1_so, agg2_so, norms, b2, 1, "so")
    p_su = _pool_call(solute_len_matrix.T, cat_su, "su")
    p_so = _pool_call(solvent_len_matrix.T, cat_so, "so")
    fcs = (fc1_W, fc1_b, fc2_W, fc2_b, fc3_W, fc3_b)
    return _mlp_call(p_su, p_so, fcs)


# speedup vs baseline: 15.4945x; 1.0331x over previous
import functools

import jax
import jax.numpy as jnp
from jax import lax
from jax.experimental import pallas as pl
from jax.experimental.pallas import tpu as pltpu
from jax.experimental.pallas import tpu_sc as plsc

N = 10000
E = 320000
NFEAT = 128
NHID = 32
NCLASS = 16
BATCH = 1024

NPAD = 10240
DUMMY = N
NC = 2
NS = 16
NW = NC * NS
CHUNK = 128
CPW = 80
CPWI = 82
EPW = CPW * CHUNK
EPAD = EPW * NW
RPT = NPAD // NS


def _mesh():
    return plsc.VectorSubcoreMesh(
        core_axis_name="c", subcore_axis_name="s", num_cores=NC, num_subcores=NS
    )


def _zero_rows(ref, nrows, width):
    z = jnp.zeros((16,), jnp.float32)

    def body(i, _):
        for h in range(width // 16):
            ref[i, pl.ds(16 * h, 16)] = z
        return 0

    lax.fori_loop(0, nrows, body, 0)


def _zero_flat(ref, nwords):
    z = jnp.zeros((16,), jnp.float32)

    def body(i, _):
        ref[pl.ds(16 * i, 16)] = z
        return 0

    lax.fori_loop(0, nwords // 16, body, 0)


def _deg_body(idx_hbm, out_hbm, d0, d1, d2, d3, idx_v, ones_v, zero_v,
              s0, s1, s2, s3):
    c = lax.axis_index("c")
    s = lax.axis_index("s")
    wid = c * NS + s
    degs = [d0, d1, d2, d3]
    sems = [s0, s1, s2, s3]

    for h in range(CHUNK // 16):
        ones_v[pl.ds(16 * h, 16)] = jnp.ones((16,), jnp.float32)
    _zero_flat(zero_v, RPT)
    for k in range(4):
        pltpu.sync_copy(zero_v, degs[k].at[pl.ds(s * RPT, RPT)])
    plsc.subcore_barrier()

    for k in range(4):
        pltpu.sync_copy(idx_hbm.at[k, wid], idx_v)
        for b in range(4):
            pltpu.async_copy(ones_v, degs[k].at[idx_v.at[b]], sems[b], add=True)

        def body(i, _, k=k):
            for b in range(4):
                j = 4 * i + b
                pltpu.make_async_copy(
                    ones_v, degs[k].at[idx_v.at[j - 4]], sems[b]
                ).wait()
                pltpu.async_copy(ones_v, degs[k].at[idx_v.at[j]], sems[b],
                                 add=True)
            return 0

        lax.fori_loop(1, CPW // 4, body, 0)
        for b in range(4):
            pltpu.make_async_copy(ones_v, degs[k].at[idx_v.at[b]], sems[b]).wait()
    plsc.subcore_barrier()

    for k in range(4):
        pltpu.sync_copy(
            degs[k].at[pl.ds(s * RPT, RPT)], out_hbm.at[c, k, pl.ds(s * RPT, RPT)]
        )


def _deg_call(idx_all):
    f = pl.kernel(
        _deg_body,
        out_type=jax.ShapeDtypeStruct((NC, 4, NPAD), jnp.float32),
        mesh=_mesh(),
        scratch_types=[
            pltpu.VMEM_SHARED((NPAD,), jnp.float32),
            pltpu.VMEM_SHARED((NPAD,), jnp.float32),
            pltpu.VMEM_SHARED((NPAD,), jnp.float32),
            pltpu.VMEM_SHARED((NPAD,), jnp.float32),
            pltpu.VMEM((CPWI, CHUNK), jnp.int32),
            pltpu.VMEM((CHUNK,), jnp.float32),
            pltpu.VMEM((RPT,), jnp.float32),
            pltpu.SemaphoreType.DMA,
            pltpu.SemaphoreType.DMA,
            pltpu.SemaphoreType.DMA,
            pltpu.SemaphoreType.DMA,
        ],
        name="gcn_degrees_sc",
        compiler_params=pltpu.CompilerParams(use_tc_tiling_on_sc=False),
    )
    return f(idx_all)


def _msg_body(D, g, idx_hbm, table_hbm, out_hbm, agg, tbl_sh, src_v, dst_v,
              r0, r1, zero_v, gs0, gs1):
    c = lax.axis_index("c")
    s = lax.axis_index("s")
    wid = c * NS + s

    _zero_rows(zero_v, RPT, D)
    pltpu.sync_copy(
        table_hbm.at[g, pl.ds(s * RPT, RPT)], tbl_sh.at[pl.ds(s * RPT, RPT)]
    )
    pltpu.sync_copy(zero_v, agg.at[pl.ds(s * RPT, RPT)])
    pltpu.sync_copy(idx_hbm.at[2 * g, wid], src_v)
    pltpu.sync_copy(idx_hbm.at[2 * g + 1, wid], dst_v)
    plsc.subcore_barrier()

    hdum = table_hbm.at[g, pl.ds(0, CHUNK)]
    pltpu.async_copy(tbl_sh.at[src_v.at[0]], r0, gs0)
    pltpu.async_copy(tbl_sh.at[src_v.at[1]], r1, gs1)

    def body(i, _):
        j0 = 2 * i
        pltpu.make_async_copy(hdum, r0, gs0).wait()
        pltpu.sync_copy(r0, agg.at[dst_v.at[j0]], add=True)
        pltpu.async_copy(tbl_sh.at[src_v.at[j0 + 2]], r0, gs0)
        pltpu.make_async_copy(hdum, r1, gs1).wait()
        pltpu.sync_copy(r1, agg.at[dst_v.at[j0 + 1]], add=True)
        pltpu.async_copy(tbl_sh.at[src_v.at[j0 + 3]], r1, gs1)
        return 0

    lax.fori_loop(0, CPW // 2, body, 0)
    pltpu.make_async_copy(hdum, r0, gs0).wait()
    pltpu.make_async_copy(hdum, r1, gs1).wait()
    plsc.subcore_barrier()

    pltpu.sync_copy(
        agg.at[pl.ds(s * RPT, RPT)], out_hbm.at[c, pl.ds(s * RPT, RPT)]
    )


def _msg_call(idx_all, table, D, g, tag):
    f = pl.kernel(
        functools.partial(_msg_body, D, g),
        out_type=jax.ShapeDtypeStruct((NC, NPAD, D), jnp.float32),
        mesh=_mesh(),
        scratch_types=[
            pltpu.VMEM_SHARED((NPAD, D), jnp.float32),
            pltpu.VMEM_SHARED((NPAD, D), jnp.float32),
            pltpu.VMEM((CPWI, CHUNK), jnp.int32),
            pltpu.VMEM((CPWI, CHUNK), jnp.int32),
            pltpu.VMEM((CHUNK, D), jnp.float32),
            pltpu.VMEM((CHUNK, D), jnp.float32),
            pltpu.VMEM((RPT, D), jnp.float32),
            pltpu.SemaphoreType.DMA,
            pltpu.SemaphoreType.DMA,
        ],
        name=f"gcn_msgpass_{tag}_sc",
        compiler_params=pltpu.CompilerParams(use_tc_tiling_on_sc=False),
    )
    return f(idx_all, table)


def _h1_body(sux, sox, w1, h1_out):
    h1_out[0] = jnp.dot(sux[...], w1[...], preferred_element_type=jnp.float32)
    h1_out[1] = jnp.dot(sox[...], w1[...], preferred_element_type=jnp.float32)


def _h1_call(sux_pad, sox_pad, w1):
    return pl.pallas_call(
        _h1_body,
        out_shape=jax.ShapeDtypeStruct((2, NPAD, NHID), jnp.float32),
        name="gcn_h1_tc",
    )(sux_pad, sox_pad, w1)


def _scale1_body(h1, degp, table_out, norms_out):
    deg = degp[0] + degp[1]
    norms = jnp.where(deg > 0.0, lax.rsqrt(jnp.maximum(deg, 1e-30)), 0.0)
    norms_out[...] = norms
    table_out[0] = h1[0] * norms[0][:, None]
    table_out[1] = h1[1] * norms[2][:, None]


def _scale1_call(h1, deg_partials):
    return pl.pallas_call(
        _scale1_body,
        out_shape=(
            jax.ShapeDtypeStruct((2, NPAD, NHID), jnp.float32),
            jax.ShapeDtypeStruct((4, NPAD), jnp.float32),
        ),
        name="gcn_scale1_tc",
    )(h1, deg_partials)


def _projB_body(g, agg1, norms, b1, w2, f1_out, table2_out):
    agg = agg1[0] + agg1[1]
    f1 = jnp.maximum(agg * norms[2 * g + 1][:, None] + b1[...][None, :], 0.0)
    f1_out[...] = f1
    h2 = jnp.dot(f1, w2[...], preferred_element_type=jnp.float32)
    table2_out[g] = h2 * norms[2 * g][:, None]
    table2_out[1 - g] = jnp.zeros((NPAD, NCLASS), jnp.float32)


def _projB_call(agg1, norms, b1, w2, g, tag):
    return pl.pallas_call(
        functools.partial(_projB_body, g),
        out_shape=(
            jax.ShapeDtypeStruct((NPAD, NHID), jnp.float32),
            jax.ShapeDtypeStruct((2, NPAD, NCLASS), jnp.float32),
        ),
        name=f"gcn_proj2_{tag}_tc",
    )(agg1, norms, b1, w2)


NB = 2048


def _fin2_body(f1_su, f1_so, agg2_su, agg2_so, norms, b2, cat_out):
    b2v = b2[...]
    f_su = (agg2_su[0] + agg2_su[1]) * norms[1][:, None] + b2v[None, :]
    f_so = (agg2_so[0] + agg2_so[1]) * norms[3][:, None] + b2v[None, :]
    cat_out[0] = jnp.concatenate([f1_su[...], f_su], axis=1)
    cat_out[1] = jnp.concatenate([f1_so[...], f_so], axis=1)


def _fin2_call(f1_su, f1_so, agg2_su, agg2_so, norms, b2):
    in_specs = [
        pl.BlockSpec((NB, NHID), lambda k: (k, 0)),
        pl.BlockSpec((NB, NHID), lambda k: (k, 0)),
        pl.BlockSpec((NC, NB, NCLASS), lambda k: (0, k, 0)),
        pl.BlockSpec((NC, NB, NCLASS), lambda k: (0, k, 0)),
        pl.BlockSpec((4, NB), lambda k: (0, k)),
        pl.BlockSpec((NCLASS,), lambda k: (0,)),
    ]
    return pl.pallas_call(
        _fin2_body,
        grid=(NPAD // NB,),
        in_specs=in_specs,
        out_specs=pl.BlockSpec((2, NB, NHID + NCLASS), lambda k: (0, k, 0)),
        out_shape=jax.ShapeDtypeStruct((2, NPAD, NHID + NCLASS), jnp.float32),
        name="gcn_finish2_tc",
    )(f1_su, f1_so, agg2_su, agg2_so, norms, b2)


MB = 128


def _pool_body(l_suT, l_soT, cat, fc1w, fc1b, fc2w, fc2b, fc3w, fc3b, out):
    dn = (((0,), (0,)), ((), ()))
    p_su = lax.dot_general(l_suT[...], cat[0, :N], dn,
                           preferred_element_type=jnp.float32)
    p_so = lax.dot_general(l_soT[...], cat[1, :N], dn,
                           preferred_element_type=jnp.float32)
    data = jnp.concatenate([p_su, p_so], axis=1)
    d1 = jnp.maximum(jnp.dot(data, fc1w[...], preferred_element_type=jnp.float32)
                     + fc1b[...][None, :], 0.0)
    d2 = jnp.maximum(jnp.dot(d1, fc2w[...], preferred_element_type=jnp.float32)
                     + fc2b[...][None, :], 0.0)
    out[...] = jnp.dot(d2, fc3w[...], preferred_element_type=jnp.float32) \
        + fc3b[...][None, :]


def _pool_call(l_suT, l_soT, cat, fcs):
    in_specs = [
        pl.BlockSpec((N, MB), lambda k: (0, k)),
        pl.BlockSpec((N, MB), lambda k: (0, k)),
        pl.BlockSpec((2, NPAD, NHID + NCLASS), lambda k: (0, 0, 0)),
        pl.BlockSpec((96, 64), lambda k: (0, 0)),
        pl.BlockSpec((64,), lambda k: (0,)),
        pl.BlockSpec((64, 16), lambda k: (0, 0)),
        pl.BlockSpec((16,), lambda k: (0,)),
        pl.BlockSpec((16, 1), lambda k: (0, 0)),
        pl.BlockSpec((1,), lambda k: (0,)),
    ]
    return pl.pallas_call(
        _pool_body,
        grid=(BATCH // MB,),
        in_specs=in_specs,
        out_specs=pl.BlockSpec((MB, 1), lambda k: (k, 0)),
        out_shape=jax.ShapeDtypeStruct((BATCH, 1), jnp.float32),
        name="gcn_pool_mlp_tc",
    )(l_suT, l_soT, cat, *fcs)


def _prep_idx(edge_index):
    pad = jnp.full((2, EPAD - E), DUMMY, jnp.int32)
    idx = jnp.concatenate([edge_index.astype(jnp.int32), pad], axis=1)
    idx = idx.reshape(2, NW, CPW, CHUNK)
    look = jnp.full((2, NW, CPWI - CPW, CHUNK), DUMMY, jnp.int32)
    return jnp.concatenate([idx, look], axis=2)


def kernel(solute_x, solute_edge_index, solvent_x, solvent_edge_index,
           solute_len_matrix, solvent_len_matrix, W1, b1, W2, b2,
           fc1_W, fc1_b, fc2_W, fc2_b, fc3_W, fc3_b):
    idx_all = jnp.concatenate(
        [_prep_idx(solute_edge_index), _prep_idx(solvent_edge_index)], axis=0
    )
    sux = jnp.pad(solute_x, ((0, NPAD - N), (0, 0)))
    sox = jnp.pad(solvent_x, ((0, NPAD - N), (0, 0)))

    deg_partials = _deg_call(idx_all)
    h1 = _h1_call(sux, sox, W1)
    table1, norms = _scale1_call(h1, deg_partials)

    agg1_su = _msg_call(idx_all, table1, NHID, 0, "l1su")
    f1_su, table2_su = _projB_call(agg1_su, norms, b1, W2, 0, "su")
    agg1_so = _msg_call(idx_all, table1, NHID, 1, "l1so")
    f1_so, table2_so = _projB_call(agg1_so, norms, b1, W2, 1, "so")

    agg2_su = _msg_call(idx_all, table2_su, NCLASS, 0, "l2su")
    agg2_so = _msg_call(idx_all, table2_so, NCLASS, 1, "l2so")

    cat = _fin2_call(f1_su, f1_so, agg2_su, agg2_so, norms, b2)
    fcs = (fc1_W, fc1_b, fc2_W, fc2_b, fc3_W, fc3_b)
    return _pool_call(solute_len_matrix.T, solvent_len_matrix.T, cat, fcs)
